# baseline (speedup 1.0000x reference)
import functools

import jax
import jax.numpy as jnp
from jax.experimental import pallas as pl
from jax.experimental.pallas import tpu as pltpu

F32 = jnp.float32
BF16 = jnp.bfloat16

_DOT_DN = (((1,), (0,)), ((), ()))
_DOT_TT = (((1,), (1,)), ((), ()))

_NBUF = 4
_NCH = 1
_BM = 256
_MLP_CHUNK = 2048


def _dot(a, b):
    return jax.lax.dot_general(a, b, _DOT_DN, preferred_element_type=F32)


def _dot_bt(a, b):
    return jax.lax.dot_general(a, b, _DOT_TT, preferred_element_type=F32)


def _mlp_chunks(aggT, sT_ref, WaT_ref, WbT_ref, b1_ref, W2T_ref, b2_ref,
                dst_ref, ncols):
    ch = min(_MLP_CHUNK, ncols)
    for p in range(ncols // ch):
        sl = pl.ds(p * ch, ch)
        hT = (_dot(WaT_ref[...], sT_ref[:, sl])
              + _dot(WbT_ref[...], aggT[:, sl])
              + b1_ref[...])
        hT = jnp.maximum(hT, 0.0)
        dst_ref[:, sl] = (_dot(W2T_ref[...], hT) + b2_ref[...]).astype(
            dst_ref.dtype)


def _body(dims, e1_hbm, e2_hbm, e3_hbm,
          xT_ref, tT_ref, cT_ref, kT_ref,
          xW_ref, xbc_ref, tW_ref, tbc_ref,
          Wa1_ref, Wb1_ref, b11_ref, W21_ref, b21_ref,
          Wa3_ref, Wb3_ref, b13_ref, W23_ref, b23_ref,
          Wa5_ref, Wb5_ref, b15_ref, W25_ref, b25_ref,
          out_ref,
          buf, sem, vT, vTb, ccT, vvT, a1, a2, a3):
    nm1, nm2, nm3, K1, K2, K3 = dims
    b1e = nm1
    b2e = nm1 + nm2
    total = nm1 + nm2 + nm3
    m = pl.program_id(0)

    def copies(i, slot):
        out = []
        for lo, hi, e_hbm, K in ((0, b1e, e1_hbm, K1),
                                 (b1e, b2e, e2_hbm, K2),
                                 (b2e, total, e3_hbm, K3)):
            kch = K // _NCH
            row = (i - lo) * _BM
            cps = [
                pltpu.make_async_copy(
                    e_hbm.at[pl.ds(row, _BM), pl.ds(j * kch, kch)],
                    buf.at[slot, :, pl.ds(j * kch, kch)],
                    sem.at[slot])
                for j in range(_NCH)
            ]
            out.append(((i >= lo) & (i < hi), cps))
        return out

    def start(i, slot):
        for cond, cps in copies(i, slot):
            @pl.when(cond)
            def _():
                for cp in cps:
                    cp.start()

    def wait(i, slot):
        for cond, cps in copies(i, slot):
            @pl.when(cond)
            def _():
                for cp in cps:
                    cp.wait()

    @pl.when(m == 0)
    def _():
        for i in range(_NBUF - 1):
            if i < total:
                start(i, i % _NBUF)

    @pl.when(m + _NBUF - 1 < total)
    def _():
        start(m + _NBUF - 1, (m + _NBUF - 1) % _NBUF)

    @pl.when(m == 0)
    def _():
        n1 = xT_ref.shape[1]
        vx = _dot(xW_ref[...], xT_ref[...]) + xbc_ref[...]
        vt = _dot(tW_ref[...], tT_ref[...]) + tbc_ref[...]
        vT[:, pl.ds(0, n1)] = vx
        vT[:, pl.ds(n1, n1)] = vt
        vTb[:, pl.ds(0, n1)] = vx.astype(BF16)
        vTb[:, pl.ds(n1, n1)] = vt.astype(BF16)

    @pl.when(m == b1e)
    def _():
        _mlp_chunks(a1, cT_ref, Wa1_ref, Wb1_ref, b11_ref, W21_ref, b21_ref,
                    ccT, cT_ref.shape[1])

    @pl.when(m == b2e)
    def _():
        _mlp_chunks(a2, vT, Wa3_ref, Wb3_ref, b13_ref, W23_ref, b23_ref,
                    vvT, vT.shape[1])

    slot = m % _NBUF
    wait(m, slot)

    for lo, _hi, rT, K, dst in ((0, b1e, vTb, K1, a1),
                                (b1e, b2e, ccT, K2, a2),
                                (b2e, total, vvT, K3, a3)):
        kch = K // _NCH

        @pl.when((m >= lo) & (m < _hi))
        def _(lo=lo, rT=rT, K=K, dst=dst, kch=kch):
            acc = _dot_bt(rT[:, pl.ds(0, kch)],
                          buf[slot, :, pl.ds(0, kch)].astype(BF16))
            for j in range(1, _NCH):
                acc += _dot_bt(rT[:, pl.ds(j * kch, kch)],
                               buf[slot, :, pl.ds(j * kch, kch)].astype(BF16))
            dst[:, pl.ds((m - lo) * _BM, _BM)] = acc

    @pl.when(m == total - 1)
    def _():
        _mlp_chunks(a3, kT_ref, Wa5_ref, Wb5_ref, b15_ref, W25_ref, b25_ref,
                    out_ref, kT_ref.shape[1])


def kernel(c, x, t, k_f, e_cv, e_vc, e_v_veh, cW, cb, xW, xb, tW, tb, kW, kb,
           f1W, f1b, f2W, f2b, f3W, f3b, f4W, f4b, f5W, f5b, f6W, f6b):
    emb = cW.shape[0]
    hid = f1W.shape[0]
    Nc, Nv = e_cv.shape
    Nk = e_v_veh.shape[0]
    nm1, nm2, nm3 = Nc // _BM, Nv // _BM, Nk // _BM
    dims = (nm1, nm2, nm3, Nv, Nc, Nv)

    W1 = f1W.T
    W1a, W1b = W1[:emb], W1[emb:]
    WaT1 = (cW.T @ W1a).T.astype(BF16)
    b1c1 = (cb @ W1a + f1b)[:, None]
    W2T1 = f2W.astype(BF16)
    b2c1 = f2b[:, None]
    WbT1 = W1b.T.astype(BF16)

    WaT3 = f3W[:, :emb].astype(BF16)
    WbT3 = f3W[:, emb:].astype(BF16)
    b1c3 = f3b[:, None]
    W2T3 = f4W.astype(BF16)
    b2c3 = f4b[:, None]

    W5 = f5W.T
    W5a, W5b = W5[:emb], W5[emb:]
    WaT5 = (kW.T @ W5b).T.astype(BF16)
    WbT5 = W5a.T.astype(BF16)
    b1c5 = (kb @ W5b + f5b)[:, None]
    W2T5 = f6W.astype(BF16)
    b2c5 = f6b[:, None]

    full = lambda arr: pl.BlockSpec(arr.shape, lambda m: tuple(
        0 for _ in arr.shape))

    small_inputs = [
        x.T, t.T, c.T, k_f.T,
        xW.astype(BF16), xb[:, None], tW.astype(BF16), tb[:, None],
        WaT1, WbT1, b1c1, W2T1, b2c1,
        WaT3, WbT3, b1c3, W2T3, b2c3,
        WaT5, WbT5, b1c5, W2T5, b2c5,
    ]

    outT = pl.pallas_call(
        functools.partial(_body, dims),
        grid=(nm1 + nm2 + nm3,),
        in_specs=[pl.BlockSpec(memory_space=pl.ANY)] * 3
                 + [full(a) for a in small_inputs],
        out_specs=pl.BlockSpec((1, Nk), lambda m: (0, 0)),
        out_shape=jax.ShapeDtypeStruct((1, Nk), F32),
        scratch_shapes=[
            pltpu.VMEM((_NBUF, _BM, Nv), F32),
            pltpu.SemaphoreType.DMA((_NBUF,)),
            pltpu.VMEM((emb, Nv), F32),
            pltpu.VMEM((emb, Nv), BF16),
            pltpu.VMEM((emb, Nc), BF16),
            pltpu.VMEM((emb, Nv), BF16),
            pltpu.VMEM((emb, Nc), F32),
            pltpu.VMEM((emb, Nv), F32),
            pltpu.VMEM((emb, Nk), F32),
        ],
        compiler_params=pltpu.CompilerParams(
            dimension_semantics=("arbitrary",)
        ),
    )(e_cv, e_vc, e_v_veh, *small_inputs)
    return outT.reshape(-1, 1)


# speedup vs baseline: 1.0187x; 1.0187x over previous
import functools

import jax
import jax.numpy as jnp
from jax.experimental import pallas as pl
from jax.experimental.pallas import tpu as pltpu

F32 = jnp.float32
BF16 = jnp.bfloat16

_DOT_DN = (((1,), (0,)), ((), ()))
_DOT_TT = (((1,), (1,)), ((), ()))

_NBUF = 3
_NCH = 4
_BM = 256
_MLP_CHUNK = 2048


def _dot(a, b):
    return jax.lax.dot_general(a, b, _DOT_DN, preferred_element_type=F32)


def _dot_bt(a, b):
    return jax.lax.dot_general(a, b, _DOT_TT, preferred_element_type=F32)


def _mlp_chunks(aggT, sT_ref, WaT_ref, WbT_ref, b1_ref, W2T_ref, b2_ref,
                dst_ref, ncols):
    ch = min(_MLP_CHUNK, ncols)
    for p in range(ncols // ch):
        sl = pl.ds(p * ch, ch)
        hT = (_dot(WaT_ref[...], sT_ref[:, sl])
              + _dot(WbT_ref[...], aggT[:, sl])
              + b1_ref[...])
        hT = jnp.maximum(hT, 0.0)
        dst_ref[:, sl] = (_dot(W2T_ref[...], hT) + b2_ref[...]).astype(
            dst_ref.dtype)


def _body(dims, e1_hbm, e2_hbm, e3_hbm,
          xT_ref, tT_ref, cT_ref, kT_ref,
          xW_ref, xbc_ref, tW_ref, tbc_ref,
          Wa1_ref, Wb1_ref, b11_ref, W21_ref, b21_ref,
          Wa3_ref, Wb3_ref, b13_ref, W23_ref, b23_ref,
          Wa5_ref, Wb5_ref, b15_ref, W25_ref, b25_ref,
          out_ref,
          buf, sem, vT, vTb, ccT, vvT, a1, a2, a3):
    nm1, nm2, nm3, K1, K2, K3 = dims
    b1e = nm1
    b2e = nm1 + nm2
    total = nm1 + nm2 + nm3
    m = pl.program_id(0)

    def copies(i, slot):
        out = []
        for lo, hi, e_hbm, K in ((0, b1e, e1_hbm, K1),
                                 (b1e, b2e, e2_hbm, K2),
                                 (b2e, total, e3_hbm, K3)):
            kch = K // _NCH
            row = (i - lo) * _BM
            cps = [
                pltpu.make_async_copy(
                    e_hbm.at[pl.ds(row, _BM), pl.ds(j * kch, kch)],
                    buf.at[slot, :, pl.ds(j * kch, kch)],
                    sem.at[slot])
                for j in range(_NCH)
            ]
            out.append(((i >= lo) & (i < hi), cps))
        return out

    def start(i, slot):
        for cond, cps in copies(i, slot):
            @pl.when(cond)
            def _():
                for cp in cps:
                    cp.start()

    def wait(i, slot):
        for cond, cps in copies(i, slot):
            @pl.when(cond)
            def _():
                for cp in cps:
                    cp.wait()

    @pl.when(m == 0)
    def _():
        for i in range(_NBUF - 1):
            if i < total:
                start(i, i % _NBUF)

    @pl.when(m + _NBUF - 1 < total)
    def _():
        start(m + _NBUF - 1, (m + _NBUF - 1) % _NBUF)

    @pl.when(m == 0)
    def _():
        n1 = xT_ref.shape[1]
        vx = _dot(xW_ref[...], xT_ref[...]) + xbc_ref[...]
        vt = _dot(tW_ref[...], tT_ref[...]) + tbc_ref[...]
        vT[:, pl.ds(0, n1)] = vx
        vT[:, pl.ds(n1, n1)] = vt
        vTb[:, pl.ds(0, n1)] = vx.astype(BF16)
        vTb[:, pl.ds(n1, n1)] = vt.astype(BF16)

    @pl.when(m == b1e)
    def _():
        _mlp_chunks(a1, cT_ref, Wa1_ref, Wb1_ref, b11_ref, W21_ref, b21_ref,
                    ccT, cT_ref.shape[1])

    @pl.when(m == b2e)
    def _():
        _mlp_chunks(a2, vT, Wa3_ref, Wb3_ref, b13_ref, W23_ref, b23_ref,
                    vvT, vT.shape[1])

    slot = m % _NBUF
    wait(m, slot)

    for lo, _hi, rT, K, dst in ((0, b1e, vTb, K1, a1),
                                (b1e, b2e, ccT, K2, a2),
                                (b2e, total, vvT, K3, a3)):
        kch = K // _NCH

        @pl.when((m >= lo) & (m < _hi))
        def _(lo=lo, rT=rT, K=K, dst=dst, kch=kch):
            acc = _dot_bt(rT[:, pl.ds(0, kch)],
                          buf[slot, :, pl.ds(0, kch)].astype(BF16))
            for j in range(1, _NCH):
                acc += _dot_bt(rT[:, pl.ds(j * kch, kch)],
                               buf[slot, :, pl.ds(j * kch, kch)].astype(BF16))
            dst[:, pl.ds((m - lo) * _BM, _BM)] = acc

    @pl.when(m == total - 1)
    def _():
        _mlp_chunks(a3, kT_ref, Wa5_ref, Wb5_ref, b15_ref, W25_ref, b25_ref,
                    out_ref, kT_ref.shape[1])


def kernel(c, x, t, k_f, e_cv, e_vc, e_v_veh, cW, cb, xW, xb, tW, tb, kW, kb,
           f1W, f1b, f2W, f2b, f3W, f3b, f4W, f4b, f5W, f5b, f6W, f6b):
    emb = cW.shape[0]
    hid = f1W.shape[0]
    Nc, Nv = e_cv.shape
    Nk = e_v_veh.shape[0]
    nm1, nm2, nm3 = Nc // _BM, Nv // _BM, Nk // _BM
    dims = (nm1, nm2, nm3, Nv, Nc, Nv)

    W1 = f1W.T
    W1a, W1b = W1[:emb], W1[emb:]
    WaT1 = (cW.T @ W1a).T.astype(BF16)
    b1c1 = (cb @ W1a + f1b)[:, None]
    W2T1 = f2W.astype(BF16)
    b2c1 = f2b[:, None]
    WbT1 = W1b.T.astype(BF16)

    WaT3 = f3W[:, :emb].astype(BF16)
    WbT3 = f3W[:, emb:].astype(BF16)
    b1c3 = f3b[:, None]
    W2T3 = f4W.astype(BF16)
    b2c3 = f4b[:, None]

    W5 = f5W.T
    W5a, W5b = W5[:emb], W5[emb:]
    WaT5 = (kW.T @ W5b).T.astype(BF16)
    WbT5 = W5a.T.astype(BF16)
    b1c5 = (kb @ W5b + f5b)[:, None]
    W2T5 = f6W.astype(BF16)
    b2c5 = f6b[:, None]

    full = lambda arr: pl.BlockSpec(arr.shape, lambda m: tuple(
        0 for _ in arr.shape))

    small_inputs = [
        x.T, t.T, c.T, k_f.T,
        xW.astype(BF16), xb[:, None], tW.astype(BF16), tb[:, None],
        WaT1, WbT1, b1c1, W2T1, b2c1,
        WaT3, WbT3, b1c3, W2T3, b2c3,
        WaT5, WbT5, b1c5, W2T5, b2c5,
    ]

    outT = pl.pallas_call(
        functools.partial(_body, dims),
        grid=(nm1 + nm2 + nm3,),
        in_specs=[pl.BlockSpec(memory_space=pl.ANY)] * 3
                 + [full(a) for a in small_inputs],
        out_specs=pl.BlockSpec((1, Nk), lambda m: (0, 0)),
        out_shape=jax.ShapeDtypeStruct((1, Nk), F32),
        scratch_shapes=[
            pltpu.VMEM((_NBUF, _BM, Nv), F32),
            pltpu.SemaphoreType.DMA((_NBUF,)),
            pltpu.VMEM((emb, Nv), F32),
            pltpu.VMEM((emb, Nv), BF16),
            pltpu.VMEM((emb, Nc), BF16),
            pltpu.VMEM((emb, Nv), BF16),
            pltpu.VMEM((emb, Nc), F32),
            pltpu.VMEM((emb, Nv), F32),
            pltpu.VMEM((emb, Nk), F32),
        ],
        compiler_params=pltpu.CompilerParams(
            dimension_semantics=("arbitrary",)
        ),
    )(e_cv, e_vc, e_v_veh, *small_inputs)
    return outT.reshape(-1, 1)
